# baseline (speedup 1.0000x reference)
import functools

import jax
import jax.numpy as jnp
from jax import lax
from jax.experimental import pallas as pl
from jax.experimental.pallas import tpu as pltpu
from jax.experimental.pallas import tpu_sc as plsc

N = 10000
E = 75000
F = 512
KC = 4
CW = 128
NSUB = 16
EB = 128
NBATCH = 38
EPT = NBATCH * EB
E_PAD = NSUB * EPT
N_ACC = 10240
STRIPE = N_ACC // NSUB
ZB = 32
PAD_DST = N + 8

@functools.cache
def _vmesh():
    return plsc.VectorSubcoreMesh(core_axis_name="c", subcore_axis_name="s")


def _agg_body(table_ref, srcs_ref, dsts_ref, zeros_ref, m0_out, m1_out,
              sidx, didx, rows_a, rows_b, zbuf, acc, sem_a, sem_b):
    cid = lax.axis_index("c")
    sid = lax.axis_index("s")

    pltpu.sync_copy(zeros_ref, zbuf)

    def run_relation(r, out_r):
        pltpu.sync_copy(srcs_ref.at[r, sid], sidx)
        pltpu.sync_copy(dsts_ref.at[r, sid], didx)

        for c in range(KC):
            @pl.loop(0, STRIPE // ZB)
            def _(z):
                pltpu.sync_copy(zbuf, acc.at[pl.ds(sid * STRIPE + z * ZB, ZB)])
            plsc.subcore_barrier()

            @pl.loop(0, NBATCH)
            def _(b):
                pltpu.sync_copy(table_ref.at[c].at[sidx.at[b]], rows_a)
                pltpu.sync_copy(rows_a, acc.at[didx.at[b]], add=True)
            plsc.subcore_barrier()
            pltpu.sync_copy(acc.at[pl.ds(sid * STRIPE, STRIPE)],
                            out_r.at[c, pl.ds(sid * STRIPE, STRIPE)])

    @pl.when(cid == 0)
    def _():
        run_relation(0, m0_out)

    @pl.when(cid == 1)
    def _():
        run_relation(1, m1_out)


@functools.cache
def _agg_kernel():
    return pl.kernel(
        _agg_body,
        out_type=[
            jax.ShapeDtypeStruct((KC, N_ACC, CW), jnp.float32),
            jax.ShapeDtypeStruct((KC, N_ACC, CW), jnp.float32),
        ],
        mesh=_vmesh(),
        scratch_types=[
            pltpu.VMEM((NBATCH, EB), jnp.int32),
            pltpu.VMEM((NBATCH, EB), jnp.int32),
            pltpu.VMEM((EB, CW), jnp.float32),
            pltpu.VMEM((EB, CW), jnp.float32),
            pltpu.VMEM((ZB, CW), jnp.float32),
            pltpu.VMEM_SHARED((N_ACC, CW), jnp.float32),
            pltpu.SemaphoreType.DMA,
            pltpu.SemaphoreType.DMA,
        ],
    )


def _deg_body(dsts_ref, ones_ref, zd_ref, deg_out,
              didx, ones_v, zd_v, dacc):
    cid = lax.axis_index("c")
    sid = lax.axis_index("s")
    pltpu.sync_copy(ones_ref, ones_v)
    pltpu.sync_copy(zd_ref, zd_v)

    def run_relation(r):
        pltpu.sync_copy(dsts_ref.at[r, sid], didx)

        @pl.loop(0, STRIPE // ZB)
        def _(z):
            pltpu.sync_copy(zd_v, dacc.at[pl.ds(sid * STRIPE + z * ZB, ZB)])
        plsc.subcore_barrier()

        @pl.loop(0, NBATCH)
        def _(b):
            pltpu.sync_copy(ones_v, dacc.at[didx.at[b]], add=True)
        plsc.subcore_barrier()
        pltpu.sync_copy(dacc.at[pl.ds(sid * STRIPE, STRIPE)],
                        deg_out.at[r, pl.ds(sid * STRIPE, STRIPE)])

    @pl.when(cid == 0)
    def _():
        run_relation(0)

    @pl.when(cid == 1)
    def _():
        run_relation(1)


@functools.cache
def _deg_kernel():
    return pl.kernel(
        _deg_body,
        out_type=jax.ShapeDtypeStruct((2, N_ACC, CW), jnp.float32),
        mesh=_vmesh(),
        scratch_types=[
            pltpu.VMEM((NBATCH, EB), jnp.int32),
            pltpu.VMEM((EB, CW), jnp.float32),
            pltpu.VMEM((ZB, CW), jnp.float32),
            pltpu.VMEM_SHARED((N_ACC, CW), jnp.float32),
        ],
    )


BN = 512


def _mm_body(relu, chunked_out, m0, m1, d0, d1, w0, w1, b0, b1, out):
    deg0 = d0[...][:, 0:1]
    deg1 = d1[...][:, 0:1]
    s0 = jnp.where(deg0 > 0, 0.5 / jnp.maximum(deg0, 1.0), 0.0)
    s1 = jnp.where(deg1 > 0, 0.5 / jnp.maximum(deg1, 1.0), 0.0)
    acc = jnp.zeros((BN, F), dtype=jnp.float32)
    for k in range(KC):
        acc += jnp.dot(m0[k] * s0, w0[k], preferred_element_type=jnp.float32)
        acc += jnp.dot(m1[k] * s1, w1[k], preferred_element_type=jnp.float32)
    acc += 0.5 * (b0[...] + b1[...])
    if relu:
        acc = jnp.maximum(acc, 0.0)
    if chunked_out:
        for k in range(KC):
            out[k] = acc[:, k * CW:(k + 1) * CW]
    else:
        out[...] = acc


def _mm_layer(m0, m1, deg0, deg1, w0, w1, b0, b1, *, relu, chunked_out):
    grid = (N_ACC // BN,)
    in_specs = [
        pl.BlockSpec((KC, BN, CW), lambda i: (0, i, 0)),
        pl.BlockSpec((KC, BN, CW), lambda i: (0, i, 0)),
        pl.BlockSpec((BN, 16), lambda i: (i, 0)),
        pl.BlockSpec((BN, 16), lambda i: (i, 0)),
        pl.BlockSpec((KC, CW, F), lambda i: (0, 0, 0)),
        pl.BlockSpec((KC, CW, F), lambda i: (0, 0, 0)),
        pl.BlockSpec((1, F), lambda i: (0, 0)),
        pl.BlockSpec((1, F), lambda i: (0, 0)),
    ]
    if chunked_out:
        out_shape = jax.ShapeDtypeStruct((KC, N_ACC, CW), jnp.float32)
        out_specs = pl.BlockSpec((KC, BN, CW), lambda i: (0, i, 0))
    else:
        out_shape = jax.ShapeDtypeStruct((N_ACC, F), jnp.float32)
        out_specs = pl.BlockSpec((BN, F), lambda i: (i, 0))
    return pl.pallas_call(
        functools.partial(_mm_body, relu, chunked_out),
        grid=grid,
        in_specs=in_specs,
        out_specs=out_specs,
        out_shape=out_shape,
    )(m0, m1, deg0, deg1, w0, w1, b0, b1)


def _fc_mm_body(gb, w, b, out):
    out[...] = jnp.dot(gb[...], w[...],
                       preferred_element_type=jnp.float32) + b[...]


BFC = 2048 // 4


def _fc_mm(gb, w, b):
    return pl.pallas_call(
        _fc_mm_body,
        grid=(2048 // BFC,),
        in_specs=[
            pl.BlockSpec((BFC, F), lambda i: (i, 0)),
            pl.BlockSpec((F, F), lambda i: (0, 0)),
            pl.BlockSpec((1, F), lambda i: (0, 0)),
        ],
        out_specs=pl.BlockSpec((BFC, F), lambda i: (i, 0)),
        out_shape=jax.ShapeDtypeStruct((2048, F), jnp.float32),
    )(gb, w, b)


BPT = 2048 // 32


def _gather_body(g_ref, bn_ref, out_ref, idx, rows):
    w = lax.axis_index("s") * 2 + lax.axis_index("c")
    pltpu.sync_copy(bn_ref.at[w], idx)
    pltpu.sync_copy(g_ref.at[idx.at[0]], rows)
    pltpu.sync_copy(rows, out_ref.at[pl.ds(w * BPT, BPT)])


@functools.cache
def _gather_kernel():
    return pl.kernel(
        _gather_body,
        out_type=jax.ShapeDtypeStruct((2048, F), jnp.float32),
        mesh=_vmesh(),
        scratch_types=[
            pltpu.VMEM((1, BPT), jnp.int32),
            pltpu.VMEM((BPT, F), jnp.float32),
        ],
    )


BPS = 2048 // NSUB


def _scatter_body(g_ref, bf_ref, bn_ref, out_ref, idx, crows, srows):
    cid = lax.axis_index("c")
    sid = lax.axis_index("s")

    @pl.when(cid == 0)
    def _():
        @pl.loop(0, STRIPE // 64)
        def _(z):
            base = sid * STRIPE + z * 64
            pltpu.sync_copy(g_ref.at[pl.ds(base, 64)], crows)
            pltpu.sync_copy(crows, out_ref.at[pl.ds(base, 64)])
        plsc.subcore_barrier()
        pltpu.sync_copy(bn_ref.at[sid], idx)
        pltpu.sync_copy(bf_ref.at[pl.ds(sid * BPS, BPS)], srows)
        pltpu.sync_copy(srows, out_ref.at[idx.at[0]])


@functools.cache
def _scatter_kernel():
    return pl.kernel(
        _scatter_body,
        out_type=jax.ShapeDtypeStruct((N_ACC, F), jnp.float32),
        mesh=_vmesh(),
        scratch_types=[
            pltpu.VMEM((1, BPS), jnp.int32),
            pltpu.VMEM((64, F), jnp.float32),
            pltpu.VMEM((BPS, F), jnp.float32),
        ],
    )


def kernel(x, edge_index_rel0, edge_index_rel1, batch_nodes,
           W1_0, b1_0, W1_1, b1_1, W2_0, b2_0, W2_1, b2_1, W_fc, b_fc):
    f32 = jnp.float32
    x4 = x.reshape(N, KC, CW).transpose(1, 0, 2)

    def pad_edges(ei):
        src = jnp.concatenate(
            [ei[0].astype(jnp.int32), jnp.zeros((E_PAD - E,), jnp.int32)])
        dst = jnp.concatenate(
            [ei[1].astype(jnp.int32),
             jnp.full((E_PAD - E,), PAD_DST, jnp.int32)])
        return src.reshape(NSUB, NBATCH, EB), dst.reshape(NSUB, NBATCH, EB)

    s0, d0 = pad_edges(edge_index_rel0)
    s1, d1 = pad_edges(edge_index_rel1)
    srcs = jnp.stack([s0, s1])
    dsts = jnp.stack([d0, d1])

    zeros_blk = jnp.zeros((ZB, CW), f32)
    ones_blk = jnp.ones((EB, CW), f32)

    w1 = jnp.stack([W1_0.reshape(KC, CW, F), W1_1.reshape(KC, CW, F)])
    w2 = jnp.stack([W2_0.reshape(KC, CW, F), W2_1.reshape(KC, CW, F)])

    deg = _deg_kernel()(dsts, ones_blk, zeros_blk)[:, :, :16]
    m0a, m1a = _agg_kernel()(x4, srcs, dsts, zeros_blk)
    h4 = _mm_layer(m0a, m1a, deg[0], deg[1], w1[0], w1[1],
                   b1_0.reshape(1, F), b1_1.reshape(1, F),
                   relu=True, chunked_out=True)

    m0b, m1b = _agg_kernel()(h4, srcs, dsts, zeros_blk)
    g = _mm_layer(m0b, m1b, deg[0], deg[1], w2[0], w2[1],
                  b2_0.reshape(1, F), b2_1.reshape(1, F),
                  relu=False, chunked_out=False)

    bn = batch_nodes.astype(jnp.int32)
    gb = _gather_kernel()(g, bn.reshape(32, 1, BPT))
    bf = _fc_mm(gb, W_fc, b_fc.reshape(1, F))
    out_full = _scatter_kernel()(g, bf, bn.reshape(NSUB, 1, BPS))
    return out_full[:N]


# speedup vs baseline: 1.8471x; 1.8471x over previous
import functools

import jax
import jax.numpy as jnp
from jax import lax
from jax.experimental import pallas as pl
from jax.experimental.pallas import tpu as pltpu
from jax.experimental.pallas import tpu_sc as plsc

N = 10000
E = 75000
F = 512
KC = 4
CW = 128
NSUB = 16
EB = 128
NBATCH = 37
EPT = NBATCH * EB
E_PAD = NSUB * EPT
N_ACC = 10240
STRIPE = N_ACC // NSUB
ZB = 64
PAD_DST = N + 8

@functools.cache
def _vmesh():
    return plsc.VectorSubcoreMesh(core_axis_name="c", subcore_axis_name="s")


def _agg_body(table_ref, srcs_ref, dsts_ref, zeros_ref, m0_out, m1_out,
              sidx, didx, rows_a, zbuf, acc):
    cid = lax.axis_index("c")
    sid = lax.axis_index("s")

    pltpu.sync_copy(zeros_ref, zbuf)

    def run_relation(r, out_r):
        pltpu.sync_copy(srcs_ref.at[r, sid], sidx)
        pltpu.sync_copy(dsts_ref.at[r, sid], didx)

        for c in range(KC):
            @pl.loop(0, STRIPE // ZB)
            def _(z):
                pltpu.sync_copy(zbuf, acc.at[pl.ds(sid * STRIPE + z * ZB, ZB)])
            plsc.subcore_barrier()

            @pl.loop(0, NBATCH)
            def _(b):
                pltpu.sync_copy(table_ref.at[c].at[sidx.at[b]], rows_a)
                pltpu.sync_copy(rows_a, acc.at[didx.at[b]], add=True)
            plsc.subcore_barrier()
            pltpu.sync_copy(acc.at[pl.ds(sid * STRIPE, STRIPE)],
                            out_r.at[c, pl.ds(sid * STRIPE, STRIPE)])

    @pl.when(cid == 0)
    def _():
        run_relation(0, m0_out)

    @pl.when(cid == 1)
    def _():
        run_relation(1, m1_out)


@functools.cache
def _agg_kernel():
    return pl.kernel(
        _agg_body,
        out_type=[
            jax.ShapeDtypeStruct((KC, N_ACC, CW), jnp.float32),
            jax.ShapeDtypeStruct((KC, N_ACC, CW), jnp.float32),
        ],
        mesh=_vmesh(),
        scratch_types=[
            pltpu.VMEM((NBATCH, EB), jnp.int32),
            pltpu.VMEM((NBATCH, EB), jnp.int32),
            pltpu.VMEM((EB, CW), jnp.float32),
            pltpu.VMEM((ZB, CW), jnp.float32),
            pltpu.VMEM_SHARED((N_ACC, CW), jnp.float32),
        ],
    )


def _deg_body(dsts_ref, ones_ref, zd_ref, deg_out,
              didx, ones_v, zd_v, dacc):
    cid = lax.axis_index("c")
    sid = lax.axis_index("s")
    pltpu.sync_copy(ones_ref, ones_v)
    pltpu.sync_copy(zd_ref, zd_v)

    def run_relation(r):
        pltpu.sync_copy(dsts_ref.at[r, sid], didx)

        @pl.loop(0, STRIPE // ZB)
        def _(z):
            pltpu.sync_copy(zd_v, dacc.at[pl.ds(sid * STRIPE + z * ZB, ZB)])
        plsc.subcore_barrier()

        @pl.loop(0, NBATCH)
        def _(b):
            pltpu.sync_copy(ones_v, dacc.at[didx.at[b]], add=True)
        plsc.subcore_barrier()
        pltpu.sync_copy(dacc.at[pl.ds(sid * STRIPE, STRIPE)],
                        deg_out.at[r, pl.ds(sid * STRIPE, STRIPE)])

    @pl.when(cid == 0)
    def _():
        run_relation(0)

    @pl.when(cid == 1)
    def _():
        run_relation(1)


@functools.cache
def _deg_kernel():
    return pl.kernel(
        _deg_body,
        out_type=jax.ShapeDtypeStruct((2, N_ACC, CW), jnp.float32),
        mesh=_vmesh(),
        scratch_types=[
            pltpu.VMEM((NBATCH, EB), jnp.int32),
            pltpu.VMEM((EB, CW), jnp.float32),
            pltpu.VMEM((ZB, CW), jnp.float32),
            pltpu.VMEM_SHARED((N_ACC, CW), jnp.float32),
        ],
    )


BN = 512


def _mm_body(relu, chunked_out, m0, m1, d0, d1, w0, w1, b0, b1, out):
    deg0 = d0[...][:, 0:1]
    deg1 = d1[...][:, 0:1]
    s0 = jnp.where(deg0 > 0, 0.5 / jnp.maximum(deg0, 1.0), 0.0)
    s1 = jnp.where(deg1 > 0, 0.5 / jnp.maximum(deg1, 1.0), 0.0)
    acc = jnp.zeros((BN, F), dtype=jnp.float32)
    for k in range(KC):
        acc += jnp.dot(m0[k] * s0, w0[k], preferred_element_type=jnp.float32)
        acc += jnp.dot(m1[k] * s1, w1[k], preferred_element_type=jnp.float32)
    acc += 0.5 * (b0[...] + b1[...])
    if relu:
        acc = jnp.maximum(acc, 0.0)
    if chunked_out:
        for k in range(KC):
            out[k] = acc[:, k * CW:(k + 1) * CW]
    else:
        out[...] = acc


def _mm_layer(m0, m1, deg0, deg1, w0, w1, b0, b1, *, relu, chunked_out):
    grid = (N_ACC // BN,)
    in_specs = [
        pl.BlockSpec((KC, BN, CW), lambda i: (0, i, 0)),
        pl.BlockSpec((KC, BN, CW), lambda i: (0, i, 0)),
        pl.BlockSpec((BN, 16), lambda i: (i, 0)),
        pl.BlockSpec((BN, 16), lambda i: (i, 0)),
        pl.BlockSpec((KC, CW, F), lambda i: (0, 0, 0)),
        pl.BlockSpec((KC, CW, F), lambda i: (0, 0, 0)),
        pl.BlockSpec((1, F), lambda i: (0, 0)),
        pl.BlockSpec((1, F), lambda i: (0, 0)),
    ]
    if chunked_out:
        out_shape = jax.ShapeDtypeStruct((KC, N_ACC, CW), jnp.float32)
        out_specs = pl.BlockSpec((KC, BN, CW), lambda i: (0, i, 0))
    else:
        out_shape = jax.ShapeDtypeStruct((N_ACC, F), jnp.float32)
        out_specs = pl.BlockSpec((BN, F), lambda i: (i, 0))
    return pl.pallas_call(
        functools.partial(_mm_body, relu, chunked_out),
        grid=grid,
        in_specs=in_specs,
        out_specs=out_specs,
        out_shape=out_shape,
    )(m0, m1, deg0, deg1, w0, w1, b0, b1)


def _fc_mm_body(gb, w, b, out):
    out[...] = jnp.dot(gb[...], w[...],
                       preferred_element_type=jnp.float32) + b[...]


BFC = 2048 // 4


def _fc_mm(gb, w, b):
    return pl.pallas_call(
        _fc_mm_body,
        grid=(2048 // BFC,),
        in_specs=[
            pl.BlockSpec((BFC, F), lambda i: (i, 0)),
            pl.BlockSpec((F, F), lambda i: (0, 0)),
            pl.BlockSpec((1, F), lambda i: (0, 0)),
        ],
        out_specs=pl.BlockSpec((BFC, F), lambda i: (i, 0)),
        out_shape=jax.ShapeDtypeStruct((2048, F), jnp.float32),
    )(gb, w, b)


BPT = 2048 // 32


def _gather_body(g_ref, bn_ref, out_ref, idx, rows):
    w = lax.axis_index("s") * 2 + lax.axis_index("c")
    pltpu.sync_copy(bn_ref.at[w], idx)
    pltpu.sync_copy(g_ref.at[idx.at[0]], rows)
    pltpu.sync_copy(rows, out_ref.at[pl.ds(w * BPT, BPT)])


@functools.cache
def _gather_kernel():
    return pl.kernel(
        _gather_body,
        out_type=jax.ShapeDtypeStruct((2048, F), jnp.float32),
        mesh=_vmesh(),
        scratch_types=[
            pltpu.VMEM((1, BPT), jnp.int32),
            pltpu.VMEM((BPT, F), jnp.float32),
        ],
    )


BPS = 2048 // NSUB


def _scatter_body(g_ref, bf_ref, bn_ref, out_ref, idx, crows, srows):
    cid = lax.axis_index("c")
    sid = lax.axis_index("s")

    @pl.when(cid == 0)
    def _():
        @pl.loop(0, STRIPE // 64)
        def _(z):
            base = sid * STRIPE + z * 64
            pltpu.sync_copy(g_ref.at[pl.ds(base, 64)], crows)
            pltpu.sync_copy(crows, out_ref.at[pl.ds(base, 64)])
        plsc.subcore_barrier()
        pltpu.sync_copy(bn_ref.at[sid], idx)
        pltpu.sync_copy(bf_ref.at[pl.ds(sid * BPS, BPS)], srows)
        pltpu.sync_copy(srows, out_ref.at[idx.at[0]])


@functools.cache
def _scatter_kernel():
    return pl.kernel(
        _scatter_body,
        out_type=jax.ShapeDtypeStruct((N_ACC, F), jnp.float32),
        mesh=_vmesh(),
        scratch_types=[
            pltpu.VMEM((1, BPS), jnp.int32),
            pltpu.VMEM((64, F), jnp.float32),
            pltpu.VMEM((BPS, F), jnp.float32),
        ],
    )


def kernel(x, edge_index_rel0, edge_index_rel1, batch_nodes,
           W1_0, b1_0, W1_1, b1_1, W2_0, b2_0, W2_1, b2_1, W_fc, b_fc):
    f32 = jnp.float32
    x4 = x.reshape(N, KC, CW).transpose(1, 0, 2)

    def pad_edges(ei):
        src = jnp.concatenate(
            [ei[0].astype(jnp.int32), jnp.zeros((E_PAD - E,), jnp.int32)])
        dst = jnp.concatenate(
            [ei[1].astype(jnp.int32),
             jnp.full((E_PAD - E,), PAD_DST, jnp.int32)])
        return src.reshape(NSUB, NBATCH, EB), dst.reshape(NSUB, NBATCH, EB)

    s0, d0 = pad_edges(edge_index_rel0)
    s1, d1 = pad_edges(edge_index_rel1)
    srcs = jnp.stack([s0, s1])
    dsts = jnp.stack([d0, d1])

    zeros_blk = jnp.zeros((ZB, CW), f32)
    ones_blk = jnp.ones((EB, CW), f32)

    w1 = jnp.stack([W1_0.reshape(KC, CW, F), W1_1.reshape(KC, CW, F)])
    w2 = jnp.stack([W2_0.reshape(KC, CW, F), W2_1.reshape(KC, CW, F)])

    deg = _deg_kernel()(dsts, ones_blk, zeros_blk)[:, :, :16]
    m0a, m1a = _agg_kernel()(x4, srcs, dsts, zeros_blk)
    h4 = _mm_layer(m0a, m1a, deg[0], deg[1], w1[0], w1[1],
                   b1_0.reshape(1, F), b1_1.reshape(1, F),
                   relu=True, chunked_out=True)

    m0b, m1b = _agg_kernel()(h4, srcs, dsts, zeros_blk)
    g = _mm_layer(m0b, m1b, deg[0], deg[1], w2[0], w2[1],
                  b2_0.reshape(1, F), b2_1.reshape(1, F),
                  relu=False, chunked_out=False)

    bn = batch_nodes.astype(jnp.int32)
    gb = _gather_kernel()(g, bn.reshape(32, 1, BPT))
    bf = _fc_mm(gb, W_fc, b_fc.reshape(1, F))
    out_full = _scatter_kernel()(g, bf, bn.reshape(NSUB, 1, BPS))
    return out_full[:N]


# speedup vs baseline: 1.8501x; 1.0016x over previous
import functools

import jax
import jax.numpy as jnp
from jax import lax
from jax.experimental import pallas as pl
from jax.experimental.pallas import tpu as pltpu
from jax.experimental.pallas import tpu_sc as plsc

N = 10000
E = 75000
F = 512
KC = 4
CW = 128
NSUB = 16
EB = 128
NBATCH = 37
EPT = NBATCH * EB
E_PAD = NSUB * EPT
N_ACC = 10240
STRIPE = N_ACC // NSUB
ZB = 64
PAD_DST = N + 8

@functools.cache
def _vmesh():
    return plsc.VectorSubcoreMesh(core_axis_name="c", subcore_axis_name="s")


def _agg_body(table_ref, srcs_ref, dsts_ref, zeros_ref, m0_out, m1_out,
              sidx, didx, rows_a, zbuf, acc):
    cid = lax.axis_index("c")
    sid = lax.axis_index("s")

    pltpu.sync_copy(zeros_ref, zbuf)

    def run_relation(r, out_r):
        pltpu.sync_copy(srcs_ref.at[r, sid], sidx)
        pltpu.sync_copy(dsts_ref.at[r, sid], didx)

        for c in range(KC):
            @pl.loop(0, STRIPE // ZB)
            def _(z):
                pltpu.sync_copy(zbuf, acc.at[pl.ds(sid * STRIPE + z * ZB, ZB)])
            plsc.subcore_barrier()

            @pl.loop(0, NBATCH)
            def _(b):
                pltpu.sync_copy(table_ref.at[c].at[sidx.at[b]], rows_a)
                pltpu.sync_copy(rows_a, acc.at[didx.at[b]], add=True)
            plsc.subcore_barrier()
            pltpu.sync_copy(acc.at[pl.ds(sid * STRIPE, STRIPE)],
                            out_r.at[c, pl.ds(sid * STRIPE, STRIPE)])

    @pl.when(cid == 0)
    def _():
        run_relation(0, m0_out)

    @pl.when(cid == 1)
    def _():
        run_relation(1, m1_out)


@functools.cache
def _agg_kernel():
    return pl.kernel(
        _agg_body,
        out_type=[
            jax.ShapeDtypeStruct((KC, N_ACC, CW), jnp.float32),
            jax.ShapeDtypeStruct((KC, N_ACC, CW), jnp.float32),
        ],
        mesh=_vmesh(),
        scratch_types=[
            pltpu.VMEM((NBATCH, EB), jnp.int32),
            pltpu.VMEM((NBATCH, EB), jnp.int32),
            pltpu.VMEM((EB, CW), jnp.float32),
            pltpu.VMEM((ZB, CW), jnp.float32),
            pltpu.VMEM_SHARED((N_ACC, CW), jnp.float32),
        ],
    )


def _deg_body(dsts_ref, ones_ref, zd_ref, deg_out,
              didx, ones_v, zd_v, dacc):
    cid = lax.axis_index("c")
    sid = lax.axis_index("s")
    pltpu.sync_copy(ones_ref, ones_v)
    pltpu.sync_copy(zd_ref, zd_v)

    def run_relation(r):
        pltpu.sync_copy(dsts_ref.at[r, sid], didx)

        @pl.loop(0, STRIPE // ZB)
        def _(z):
            pltpu.sync_copy(zd_v, dacc.at[pl.ds(sid * STRIPE + z * ZB, ZB)])
        plsc.subcore_barrier()

        @pl.loop(0, NBATCH)
        def _(b):
            pltpu.sync_copy(ones_v, dacc.at[didx.at[b]], add=True)
        plsc.subcore_barrier()
        pltpu.sync_copy(dacc.at[pl.ds(sid * STRIPE, STRIPE)],
                        deg_out.at[r, pl.ds(sid * STRIPE, STRIPE)])

    @pl.when(cid == 0)
    def _():
        run_relation(0)

    @pl.when(cid == 1)
    def _():
        run_relation(1)


@functools.cache
def _deg_kernel():
    return pl.kernel(
        _deg_body,
        out_type=jax.ShapeDtypeStruct((2, N_ACC, CW), jnp.float32),
        mesh=_vmesh(),
        scratch_types=[
            pltpu.VMEM((NBATCH, EB), jnp.int32),
            pltpu.VMEM((EB, CW), jnp.float32),
            pltpu.VMEM((ZB, CW), jnp.float32),
            pltpu.VMEM_SHARED((N_ACC, CW), jnp.float32),
        ],
    )


BN = 512


def _mm_body(relu, chunked_out, m0, m1, d0, d1, w0, w1, b0, b1, out):
    deg0 = d0[...][:, 0:1]
    deg1 = d1[...][:, 0:1]
    s0 = jnp.where(deg0 > 0, 0.5 / jnp.maximum(deg0, 1.0), 0.0)
    s1 = jnp.where(deg1 > 0, 0.5 / jnp.maximum(deg1, 1.0), 0.0)
    acc = jnp.zeros((BN, F), dtype=jnp.float32)
    for k in range(KC):
        acc += jnp.dot((m0[k] * s0).astype(jnp.bfloat16), w0[k],
                       preferred_element_type=jnp.float32)
        acc += jnp.dot((m1[k] * s1).astype(jnp.bfloat16), w1[k],
                       preferred_element_type=jnp.float32)
    acc += 0.5 * (b0[...] + b1[...])
    if relu:
        acc = jnp.maximum(acc, 0.0)
    if chunked_out:
        for k in range(KC):
            out[k] = acc[:, k * CW:(k + 1) * CW]
    else:
        out[...] = acc


def _mm_layer(m0, m1, deg0, deg1, w0, w1, b0, b1, *, relu, chunked_out):
    grid = (N_ACC // BN,)
    in_specs = [
        pl.BlockSpec((KC, BN, CW), lambda i: (0, i, 0)),
        pl.BlockSpec((KC, BN, CW), lambda i: (0, i, 0)),
        pl.BlockSpec((BN, 16), lambda i: (i, 0)),
        pl.BlockSpec((BN, 16), lambda i: (i, 0)),
        pl.BlockSpec((KC, CW, F), lambda i: (0, 0, 0)),
        pl.BlockSpec((KC, CW, F), lambda i: (0, 0, 0)),
        pl.BlockSpec((1, F), lambda i: (0, 0)),
        pl.BlockSpec((1, F), lambda i: (0, 0)),
    ]
    if chunked_out:
        out_shape = jax.ShapeDtypeStruct((KC, N_ACC, CW), jnp.float32)
        out_specs = pl.BlockSpec((KC, BN, CW), lambda i: (0, i, 0))
    else:
        out_shape = jax.ShapeDtypeStruct((N_ACC, F), jnp.float32)
        out_specs = pl.BlockSpec((BN, F), lambda i: (i, 0))
    return pl.pallas_call(
        functools.partial(_mm_body, relu, chunked_out),
        grid=grid,
        in_specs=in_specs,
        out_specs=out_specs,
        out_shape=out_shape,
    )(m0, m1, deg0, deg1, w0, w1, b0, b1)


def _fc_mm_body(gb, w, b, out):
    out[...] = jnp.dot(gb[...], w[...],
                       preferred_element_type=jnp.float32) + b[...]


BFC = 2048 // 4


def _fc_mm(gb, w, b):
    return pl.pallas_call(
        _fc_mm_body,
        grid=(2048 // BFC,),
        in_specs=[
            pl.BlockSpec((BFC, F), lambda i: (i, 0)),
            pl.BlockSpec((F, F), lambda i: (0, 0)),
            pl.BlockSpec((1, F), lambda i: (0, 0)),
        ],
        out_specs=pl.BlockSpec((BFC, F), lambda i: (i, 0)),
        out_shape=jax.ShapeDtypeStruct((2048, F), jnp.float32),
    )(gb, w, b)


BPT = 2048 // 32


def _gather_body(g_ref, bn_ref, out_ref, idx, rows):
    w = lax.axis_index("s") * 2 + lax.axis_index("c")
    pltpu.sync_copy(bn_ref.at[w], idx)
    pltpu.sync_copy(g_ref.at[idx.at[0]], rows)
    pltpu.sync_copy(rows, out_ref.at[pl.ds(w * BPT, BPT)])


@functools.cache
def _gather_kernel():
    return pl.kernel(
        _gather_body,
        out_type=jax.ShapeDtypeStruct((2048, F), jnp.float32),
        mesh=_vmesh(),
        scratch_types=[
            pltpu.VMEM((1, BPT), jnp.int32),
            pltpu.VMEM((BPT, F), jnp.float32),
        ],
    )


BPS = 2048 // NSUB


def _scatter_body(g_ref, bf_ref, bn_ref, out_ref, idx, crows, srows):
    cid = lax.axis_index("c")
    sid = lax.axis_index("s")

    @pl.when(cid == 0)
    def _():
        @pl.loop(0, STRIPE // 64)
        def _(z):
            base = sid * STRIPE + z * 64
            pltpu.sync_copy(g_ref.at[pl.ds(base, 64)], crows)
            pltpu.sync_copy(crows, out_ref.at[pl.ds(base, 64)])
        plsc.subcore_barrier()
        pltpu.sync_copy(bn_ref.at[sid], idx)
        pltpu.sync_copy(bf_ref.at[pl.ds(sid * BPS, BPS)], srows)
        pltpu.sync_copy(srows, out_ref.at[idx.at[0]])


@functools.cache
def _scatter_kernel():
    return pl.kernel(
        _scatter_body,
        out_type=jax.ShapeDtypeStruct((N_ACC, F), jnp.float32),
        mesh=_vmesh(),
        scratch_types=[
            pltpu.VMEM((1, BPS), jnp.int32),
            pltpu.VMEM((64, F), jnp.float32),
            pltpu.VMEM((BPS, F), jnp.float32),
        ],
    )


def kernel(x, edge_index_rel0, edge_index_rel1, batch_nodes,
           W1_0, b1_0, W1_1, b1_1, W2_0, b2_0, W2_1, b2_1, W_fc, b_fc):
    f32 = jnp.float32
    x4 = x.reshape(N, KC, CW).transpose(1, 0, 2)

    def pad_edges(ei):
        src = jnp.concatenate(
            [ei[0].astype(jnp.int32), jnp.zeros((E_PAD - E,), jnp.int32)])
        dst = jnp.concatenate(
            [ei[1].astype(jnp.int32),
             jnp.full((E_PAD - E,), PAD_DST, jnp.int32)])
        return src.reshape(NSUB, NBATCH, EB), dst.reshape(NSUB, NBATCH, EB)

    s0, d0 = pad_edges(edge_index_rel0)
    s1, d1 = pad_edges(edge_index_rel1)
    srcs = jnp.stack([s0, s1])
    dsts = jnp.stack([d0, d1])

    zeros_blk = jnp.zeros((ZB, CW), f32)
    ones_blk = jnp.ones((EB, CW), f32)

    bf16 = jnp.bfloat16
    w1 = jnp.stack([W1_0.reshape(KC, CW, F), W1_1.reshape(KC, CW, F)]).astype(bf16)
    w2 = jnp.stack([W2_0.reshape(KC, CW, F), W2_1.reshape(KC, CW, F)]).astype(bf16)

    deg = _deg_kernel()(dsts, ones_blk, zeros_blk)[:, :, :16]
    m0a, m1a = _agg_kernel()(x4, srcs, dsts, zeros_blk)
    h4 = _mm_layer(m0a, m1a, deg[0], deg[1], w1[0], w1[1],
                   b1_0.reshape(1, F), b1_1.reshape(1, F),
                   relu=True, chunked_out=True)

    m0b, m1b = _agg_kernel()(h4, srcs, dsts, zeros_blk)
    g = _mm_layer(m0b, m1b, deg[0], deg[1], w2[0], w2[1],
                  b2_0.reshape(1, F), b2_1.reshape(1, F),
                  relu=False, chunked_out=False)

    bn = batch_nodes.astype(jnp.int32)
    gb = _gather_kernel()(g, bn.reshape(32, 1, BPT))
    bf = _fc_mm(gb, W_fc, b_fc.reshape(1, F))
    out_full = _scatter_kernel()(g, bf, bn.reshape(NSUB, 1, BPS))
    return out_full[:N]
